# baseline (speedup 1.0000x reference)
import jax
import jax.numpy as jnp
from jax import lax
from jax.experimental import pallas as pl
from jax.experimental.pallas import tpu as pltpu

_B, _T, _I, _H = 64, 2048, 128, 256
_TC = 64
_NT = _T // _TC
_G = 4 * _H
_NS = 2
_SB = _B // _NS


def _gates(g, c):
    i = jax.nn.sigmoid(g[:, 0 * _H:1 * _H])
    f = jax.nn.sigmoid(g[:, 1 * _H:2 * _H])
    gg = jnp.tanh(g[:, 2 * _H:3 * _H])
    o = jax.nn.sigmoid(g[:, 3 * _H:4 * _H])
    c_new = f * c + i * gg
    h_new = o * jnp.tanh(c_new)
    return h_new, c_new


def _lstm_kernel(x_ref, wih0_ref, whh0_ref, b0_ref, wih1_ref, whh1_ref,
                 b1_ref, m0_ref, m1_ref,
                 out_ref, hn_ref, cn_ref,
                 xw_ref, h1buf_ref, h0s, c0s, h1s, c1s):
    t = pl.program_id(0)

    @pl.when(t == 0)
    def _():
        h0s[...] = jnp.zeros_like(h0s)
        c0s[...] = jnp.zeros_like(c0s)
        h1s[...] = jnp.zeros_like(h1s)
        c1s[...] = jnp.zeros_like(c1s)

    xb = x_ref[...].reshape(_TC * _B, _I)
    xw = jnp.dot(xb, wih0_ref[...], preferred_element_type=jnp.float32)
    xw_ref[...] = (xw + b0_ref[...]).reshape(_TC, _B, _G)

    whh0 = whh0_ref[...]
    m0 = [m0_ref[i * _SB:(i + 1) * _SB, :] for i in range(_NS)]

    def step0(s, carry):
        hs, cs = carry[:_NS], carry[_NS:]
        gs = [xw_ref[s, i * _SB:(i + 1) * _SB, :]
              + jnp.dot(hs[i], whh0, preferred_element_type=jnp.float32)
              for i in range(_NS)]
        new = [_gates(gs[i], cs[i]) for i in range(_NS)]
        for i in range(_NS):
            h1buf_ref[s, i * _SB:(i + 1) * _SB, :] = new[i][0] * m0[i]
        return tuple(n[0] for n in new) + tuple(n[1] for n in new)

    init0 = (tuple(h0s[i * _SB:(i + 1) * _SB, :] for i in range(_NS))
             + tuple(c0s[i * _SB:(i + 1) * _SB, :] for i in range(_NS)))
    fin0 = lax.fori_loop(0, _TC, step0, init0)
    for i in range(_NS):
        h0s[i * _SB:(i + 1) * _SB, :] = fin0[i]
        c0s[i * _SB:(i + 1) * _SB, :] = fin0[_NS + i]

    hb = h1buf_ref[...].reshape(_TC * _B, _H)
    xw = jnp.dot(hb, wih1_ref[...], preferred_element_type=jnp.float32)
    xw_ref[...] = (xw + b1_ref[...]).reshape(_TC, _B, _G)

    whh1 = whh1_ref[...]
    m1 = [m1_ref[i * _SB:(i + 1) * _SB, :] for i in range(_NS)]

    def step1(s, carry):
        hs, cs = carry[:_NS], carry[_NS:]
        gs = [xw_ref[s, i * _SB:(i + 1) * _SB, :]
              + jnp.dot(hs[i], whh1, preferred_element_type=jnp.float32)
              for i in range(_NS)]
        new = [_gates(gs[i], cs[i]) for i in range(_NS)]
        for i in range(_NS):
            out_ref[s, i * _SB:(i + 1) * _SB, :] = new[i][0] * m1[i]
        return tuple(n[0] for n in new) + tuple(n[1] for n in new)

    init1 = (tuple(h1s[i * _SB:(i + 1) * _SB, :] for i in range(_NS))
             + tuple(c1s[i * _SB:(i + 1) * _SB, :] for i in range(_NS)))
    fin1 = lax.fori_loop(0, _TC, step1, init1)
    for i in range(_NS):
        h1s[i * _SB:(i + 1) * _SB, :] = fin1[i]
        c1s[i * _SB:(i + 1) * _SB, :] = fin1[_NS + i]

    @pl.when(t == _NT - 1)
    def _():
        hn_ref[0] = h0s[...]
        hn_ref[1] = h1s[...]
        cn_ref[0] = c0s[...]
        cn_ref[1] = c1s[...]


def kernel(x, W_ih0, W_hh0, b_ih0, b_hh0, W_ih1, W_hh1, b_ih1, b_hh1,
           mask0, mask1):
    x_tm = jnp.swapaxes(x, 0, 1)
    wih0T = W_ih0.T
    whh0T = W_hh0.T
    b0 = (b_ih0 + b_hh0).reshape(1, _G)
    wih1T = W_ih1.T
    whh1T = W_hh1.T
    b1 = (b_ih1 + b_hh1).reshape(1, _G)

    out_tm, hn, cn = pl.pallas_call(
        _lstm_kernel,
        grid=(_NT,),
        in_specs=[
            pl.BlockSpec((_TC, _B, _I), lambda t: (t, 0, 0)),
            pl.BlockSpec((_I, _G), lambda t: (0, 0)),
            pl.BlockSpec((_H, _G), lambda t: (0, 0)),
            pl.BlockSpec((1, _G), lambda t: (0, 0)),
            pl.BlockSpec((_H, _G), lambda t: (0, 0)),
            pl.BlockSpec((_H, _G), lambda t: (0, 0)),
            pl.BlockSpec((1, _G), lambda t: (0, 0)),
            pl.BlockSpec((_B, _H), lambda t: (0, 0)),
            pl.BlockSpec((_B, _H), lambda t: (0, 0)),
        ],
        out_specs=[
            pl.BlockSpec((_TC, _B, _H), lambda t: (t, 0, 0)),
            pl.BlockSpec((2, _B, _H), lambda t: (0, 0, 0)),
            pl.BlockSpec((2, _B, _H), lambda t: (0, 0, 0)),
        ],
        out_shape=[
            jax.ShapeDtypeStruct((_T, _B, _H), jnp.float32),
            jax.ShapeDtypeStruct((2, _B, _H), jnp.float32),
            jax.ShapeDtypeStruct((2, _B, _H), jnp.float32),
        ],
        scratch_shapes=[
            pltpu.VMEM((_TC, _B, _G), jnp.float32),
            pltpu.VMEM((_TC, _B, _H), jnp.float32),
            pltpu.VMEM((_B, _H), jnp.float32),
            pltpu.VMEM((_B, _H), jnp.float32),
            pltpu.VMEM((_B, _H), jnp.float32),
            pltpu.VMEM((_B, _H), jnp.float32),
        ],
        compiler_params=pltpu.CompilerParams(
            dimension_semantics=("arbitrary",),
            vmem_limit_bytes=52 * 1024 * 1024,
        ),
        name="vd_lstm2",
    )(x_tm, wih0T, whh0T, b0, wih1T, whh1T, b1, mask0, mask1)

    out = jnp.swapaxes(out_tm, 0, 1)
    return out, (hn, cn)


# speedup vs baseline: 1.2794x; 1.2794x over previous
import jax
import jax.numpy as jnp
from jax import lax
from jax.experimental import pallas as pl
from jax.experimental.pallas import tpu as pltpu

_B, _T, _I, _H = 64, 2048, 128, 256
_TC = 64
_NT = _T // _TC
_G = 4 * _H


def _gates(g, c):
    i = jax.nn.sigmoid(g[:, 0 * _H:1 * _H])
    f = jax.nn.sigmoid(g[:, 1 * _H:2 * _H])
    gg = jnp.tanh(g[:, 2 * _H:3 * _H])
    o = jax.nn.sigmoid(g[:, 3 * _H:4 * _H])
    c_new = f * c + i * gg
    h_new = o * jnp.tanh(c_new)
    return h_new, c_new


def _lstm_kernel(x_ref, wih0_ref, whh0_ref, b0_ref, wih1_ref, whh1_ref,
                 b1_ref, m0_ref, m1_ref,
                 out_ref, hn_ref, cn_ref,
                 xw_ref, h1buf_ref, obuf_ref, h0s, c0s, h1s, c1s):
    t = pl.program_id(0)

    @pl.when(t == 0)
    def _():
        h0s[...] = jnp.zeros_like(h0s)
        c0s[...] = jnp.zeros_like(c0s)
        h1s[...] = jnp.zeros_like(h1s)
        c1s[...] = jnp.zeros_like(c1s)

    xb = jnp.swapaxes(x_ref[...], 0, 1).reshape(_TC * _B, _I)
    xw = jnp.dot(xb, wih0_ref[...], preferred_element_type=jnp.float32)
    xw_ref[...] = (xw + b0_ref[...]).reshape(_TC, _B, _G)

    whh0 = whh0_ref[...]
    m0 = m0_ref[...]

    def step0(s, carry):
        h, c = carry
        g = xw_ref[s] + jnp.dot(h, whh0, preferred_element_type=jnp.float32)
        h, c = _gates(g, c)
        h1buf_ref[s] = h * m0
        return (h, c)

    h0, c0 = lax.fori_loop(0, _TC, step0, (h0s[...], c0s[...]))
    h0s[...] = h0
    c0s[...] = c0

    hb = h1buf_ref[...].reshape(_TC * _B, _H)
    xw = jnp.dot(hb, wih1_ref[...], preferred_element_type=jnp.float32)
    xw_ref[...] = (xw + b1_ref[...]).reshape(_TC, _B, _G)

    whh1 = whh1_ref[...]
    m1 = m1_ref[...]

    def step1(s, carry):
        h, c = carry
        g = xw_ref[s] + jnp.dot(h, whh1, preferred_element_type=jnp.float32)
        h, c = _gates(g, c)
        obuf_ref[s] = h * m1
        return (h, c)

    h1, c1 = lax.fori_loop(0, _TC, step1, (h1s[...], c1s[...]))
    h1s[...] = h1
    c1s[...] = c1

    out_ref[...] = jnp.swapaxes(obuf_ref[...], 0, 1)

    @pl.when(t == _NT - 1)
    def _():
        hn_ref[0] = h0
        hn_ref[1] = h1
        cn_ref[0] = c0
        cn_ref[1] = c1


def kernel(x, W_ih0, W_hh0, b_ih0, b_hh0, W_ih1, W_hh1, b_ih1, b_hh1,
           mask0, mask1):
    wih0T = W_ih0.T
    whh0T = W_hh0.T
    b0 = (b_ih0 + b_hh0).reshape(1, _G)
    wih1T = W_ih1.T
    whh1T = W_hh1.T
    b1 = (b_ih1 + b_hh1).reshape(1, _G)

    out, hn, cn = pl.pallas_call(
        _lstm_kernel,
        grid=(_NT,),
        in_specs=[
            pl.BlockSpec((_B, _TC, _I), lambda t: (0, t, 0)),
            pl.BlockSpec((_I, _G), lambda t: (0, 0)),
            pl.BlockSpec((_H, _G), lambda t: (0, 0)),
            pl.BlockSpec((1, _G), lambda t: (0, 0)),
            pl.BlockSpec((_H, _G), lambda t: (0, 0)),
            pl.BlockSpec((_H, _G), lambda t: (0, 0)),
            pl.BlockSpec((1, _G), lambda t: (0, 0)),
            pl.BlockSpec((_B, _H), lambda t: (0, 0)),
            pl.BlockSpec((_B, _H), lambda t: (0, 0)),
        ],
        out_specs=[
            pl.BlockSpec((_B, _TC, _H), lambda t: (0, t, 0)),
            pl.BlockSpec((2, _B, _H), lambda t: (0, 0, 0)),
            pl.BlockSpec((2, _B, _H), lambda t: (0, 0, 0)),
        ],
        out_shape=[
            jax.ShapeDtypeStruct((_B, _T, _H), jnp.float32),
            jax.ShapeDtypeStruct((2, _B, _H), jnp.float32),
            jax.ShapeDtypeStruct((2, _B, _H), jnp.float32),
        ],
        scratch_shapes=[
            pltpu.VMEM((_TC, _B, _G), jnp.float32),
            pltpu.VMEM((_TC, _B, _H), jnp.float32),
            pltpu.VMEM((_TC, _B, _H), jnp.float32),
            pltpu.VMEM((_B, _H), jnp.float32),
            pltpu.VMEM((_B, _H), jnp.float32),
            pltpu.VMEM((_B, _H), jnp.float32),
            pltpu.VMEM((_B, _H), jnp.float32),
        ],
        compiler_params=pltpu.CompilerParams(
            dimension_semantics=("arbitrary",),
            vmem_limit_bytes=52 * 1024 * 1024,
        ),
        name="vd_lstm2",
    )(x, wih0T, whh0T, b0, wih1T, whh1T, b1, mask0, mask1)

    return out, (hn, cn)


# speedup vs baseline: 1.5428x; 1.2059x over previous
import jax
import jax.numpy as jnp
from jax import lax
from jax.experimental import pallas as pl
from jax.experimental.pallas import tpu as pltpu

_B, _T, _I, _H = 64, 2048, 128, 256
_TC = 32
_NT = _T // _TC
_G = 4 * _H


def _gates(g, c):
    i = jax.nn.sigmoid(g[:, 0 * _H:1 * _H])
    f = jax.nn.sigmoid(g[:, 1 * _H:2 * _H])
    gg = jnp.tanh(g[:, 2 * _H:3 * _H])
    o = jax.nn.sigmoid(g[:, 3 * _H:4 * _H])
    c_new = f * c + i * gg
    h_new = o * jnp.tanh(c_new)
    return h_new, c_new


def _lstm_kernel(x_ref, wih0_ref, whh0_ref, b0_ref, wih1_ref, whh1_ref,
                 b1_ref, m0_ref, m1_ref,
                 out_ref, hn_ref, cn_ref,
                 xw0_ref, xw1_ref, h1buf_ref, obuf_ref,
                 h0s, c0s, h1s, c1s):
    t = pl.program_id(0)
    cur = t % 2
    prev = (t + 1) % 2

    @pl.when(t == 0)
    def _():
        h0s[...] = jnp.zeros_like(h0s)
        c0s[...] = jnp.zeros_like(c0s)
        h1s[...] = jnp.zeros_like(h1s)
        c1s[...] = jnp.zeros_like(c1s)
        h1buf_ref[1] = jnp.zeros_like(h1buf_ref[1])

    hb = h1buf_ref[prev].reshape(_TC * _B, _H)
    xw1 = jnp.dot(hb, wih1_ref[...], preferred_element_type=jnp.float32)
    xw1_ref[...] = (xw1 + b1_ref[...]).reshape(_TC, _B, _G)

    xb = jnp.swapaxes(x_ref[...], 0, 1).reshape(_TC * _B, _I)
    xw0 = jnp.dot(xb, wih0_ref[...], preferred_element_type=jnp.float32)
    xw0_ref[...] = (xw0 + b0_ref[...]).reshape(_TC, _B, _G)

    whh0 = whh0_ref[...]
    whh1 = whh1_ref[...]
    m0 = m0_ref[...]
    m1 = m1_ref[...]

    def step(s, carry):
        h0, c0, h1, c1 = carry
        g0 = xw0_ref[s] + jnp.dot(h0, whh0, preferred_element_type=jnp.float32)
        g1 = xw1_ref[s] + jnp.dot(h1, whh1, preferred_element_type=jnp.float32)
        h0, c0 = _gates(g0, c0)
        h1, c1 = _gates(g1, c1)
        h1buf_ref[cur, s] = h0 * m0
        obuf_ref[s] = h1 * m1
        return (h0, c0, h1, c1)

    h0, c0, h1, c1 = lax.fori_loop(
        0, _TC, step, (h0s[...], c0s[...], h1s[...], c1s[...]))

    @pl.when(t < _NT)
    def _():
        h0s[...] = h0
        c0s[...] = c0

    @pl.when(t > 0)
    def _():
        h1s[...] = h1
        c1s[...] = c1

    out_ref[...] = jnp.swapaxes(obuf_ref[...], 0, 1)

    @pl.when(t == _NT)
    def _():
        hn_ref[0] = h0s[...]
        hn_ref[1] = h1
        cn_ref[0] = c0s[...]
        cn_ref[1] = c1


def kernel(x, W_ih0, W_hh0, b_ih0, b_hh0, W_ih1, W_hh1, b_ih1, b_hh1,
           mask0, mask1):
    wih0T = W_ih0.T
    whh0T = W_hh0.T
    b0 = (b_ih0 + b_hh0).reshape(1, _G)
    wih1T = W_ih1.T
    whh1T = W_hh1.T
    b1 = (b_ih1 + b_hh1).reshape(1, _G)

    out, hn, cn = pl.pallas_call(
        _lstm_kernel,
        grid=(_NT + 1,),
        in_specs=[
            pl.BlockSpec((_B, _TC, _I),
                         lambda t: (0, jnp.minimum(t, _NT - 1), 0)),
            pl.BlockSpec((_I, _G), lambda t: (0, 0)),
            pl.BlockSpec((_H, _G), lambda t: (0, 0)),
            pl.BlockSpec((1, _G), lambda t: (0, 0)),
            pl.BlockSpec((_H, _G), lambda t: (0, 0)),
            pl.BlockSpec((_H, _G), lambda t: (0, 0)),
            pl.BlockSpec((1, _G), lambda t: (0, 0)),
            pl.BlockSpec((_B, _H), lambda t: (0, 0)),
            pl.BlockSpec((_B, _H), lambda t: (0, 0)),
        ],
        out_specs=[
            pl.BlockSpec((_B, _TC, _H),
                         lambda t: (0, jnp.maximum(t - 1, 0), 0)),
            pl.BlockSpec((2, _B, _H), lambda t: (0, 0, 0)),
            pl.BlockSpec((2, _B, _H), lambda t: (0, 0, 0)),
        ],
        out_shape=[
            jax.ShapeDtypeStruct((_B, _T, _H), jnp.float32),
            jax.ShapeDtypeStruct((2, _B, _H), jnp.float32),
            jax.ShapeDtypeStruct((2, _B, _H), jnp.float32),
        ],
        scratch_shapes=[
            pltpu.VMEM((_TC, _B, _G), jnp.float32),
            pltpu.VMEM((_TC, _B, _G), jnp.float32),
            pltpu.VMEM((2, _TC, _B, _H), jnp.float32),
            pltpu.VMEM((_TC, _B, _H), jnp.float32),
            pltpu.VMEM((_B, _H), jnp.float32),
            pltpu.VMEM((_B, _H), jnp.float32),
            pltpu.VMEM((_B, _H), jnp.float32),
            pltpu.VMEM((_B, _H), jnp.float32),
        ],
        compiler_params=pltpu.CompilerParams(
            dimension_semantics=("arbitrary",),
            vmem_limit_bytes=52 * 1024 * 1024,
        ),
        name="vd_lstm2",
    )(x, wih0T, whh0T, b0, wih1T, whh1T, b1, mask0, mask1)

    return out, (hn, cn)


# speedup vs baseline: 1.8440x; 1.1952x over previous
import jax
import jax.numpy as jnp
from jax import lax
from jax.experimental import pallas as pl
from jax.experimental.pallas import tpu as pltpu

_B, _T, _I, _H = 64, 2048, 128, 256
_TC = 32
_NT = _T // _TC
_G = 4 * _H


def _gates(g, c):
    i = jax.nn.sigmoid(g[:, 0 * _H:1 * _H])
    f = jax.nn.sigmoid(g[:, 1 * _H:2 * _H])
    gg = jnp.tanh(g[:, 2 * _H:3 * _H])
    o = jax.nn.sigmoid(g[:, 3 * _H:4 * _H])
    c_new = f * c + i * gg
    h_new = o * jnp.tanh(c_new)
    return h_new, c_new


def _lstm_kernel(x_ref, wih0_ref, whh0_ref, b0_ref, wih1_ref, whh1_ref,
                 b1_ref, m0_ref, m1_ref,
                 out_ref, hn_ref, cn_ref,
                 xw0_ref, xw1_ref, h1buf_ref, obuf_ref,
                 h0s, c0s, h1s, c1s):
    t = pl.program_id(0)
    cur = t % 2
    prev = (t + 1) % 2

    @pl.when(t == 0)
    def _():
        h0s[...] = jnp.zeros_like(h0s)
        c0s[...] = jnp.zeros_like(c0s)
        h1s[...] = jnp.zeros_like(h1s)
        c1s[...] = jnp.zeros_like(c1s)
        h1buf_ref[1] = jnp.zeros_like(h1buf_ref[1])

    hb = h1buf_ref[prev].reshape(_TC * _B, _H)
    xw1 = jnp.dot(hb, wih1_ref[...], preferred_element_type=jnp.float32)
    xw1_ref[...] = (xw1 + b1_ref[...]).reshape(_TC, _B, _G)

    xb = jnp.swapaxes(x_ref[...], 0, 1).reshape(_TC * _B, _I)
    xw0 = jnp.dot(xb, wih0_ref[...], preferred_element_type=jnp.float32)
    xw0_ref[...] = (xw0 + b0_ref[...]).reshape(_TC, _B, _G)

    whh0 = whh0_ref[...]
    whh1 = whh1_ref[...]
    m0 = m0_ref[...]
    m1 = m1_ref[...]

    def step(s, carry):
        h0, c0, h1, c1 = carry
        g0 = xw0_ref[s] + jnp.dot(h0, whh0, preferred_element_type=jnp.float32)
        g1 = xw1_ref[s] + jnp.dot(h1, whh1, preferred_element_type=jnp.float32)
        h0, c0 = _gates(g0, c0)
        h1, c1 = _gates(g1, c1)
        h1buf_ref[cur, s] = h0 * m0
        obuf_ref[s] = h1 * m1
        return (h0, c0, h1, c1)

    h0, c0, h1, c1 = lax.fori_loop(
        0, _TC, step, (h0s[...], c0s[...], h1s[...], c1s[...]), unroll=2)

    @pl.when(t < _NT)
    def _():
        h0s[...] = h0
        c0s[...] = c0

    @pl.when(t > 0)
    def _():
        h1s[...] = h1
        c1s[...] = c1

    out_ref[...] = jnp.swapaxes(obuf_ref[...], 0, 1)

    @pl.when(t == _NT)
    def _():
        hn_ref[0] = h0s[...]
        hn_ref[1] = h1
        cn_ref[0] = c0s[...]
        cn_ref[1] = c1


def kernel(x, W_ih0, W_hh0, b_ih0, b_hh0, W_ih1, W_hh1, b_ih1, b_hh1,
           mask0, mask1):
    wih0T = W_ih0.T
    whh0T = W_hh0.T
    b0 = (b_ih0 + b_hh0).reshape(1, _G)
    wih1T = W_ih1.T
    whh1T = W_hh1.T
    b1 = (b_ih1 + b_hh1).reshape(1, _G)

    out, hn, cn = pl.pallas_call(
        _lstm_kernel,
        grid=(_NT + 1,),
        in_specs=[
            pl.BlockSpec((_B, _TC, _I),
                         lambda t: (0, jnp.minimum(t, _NT - 1), 0)),
            pl.BlockSpec((_I, _G), lambda t: (0, 0)),
            pl.BlockSpec((_H, _G), lambda t: (0, 0)),
            pl.BlockSpec((1, _G), lambda t: (0, 0)),
            pl.BlockSpec((_H, _G), lambda t: (0, 0)),
            pl.BlockSpec((_H, _G), lambda t: (0, 0)),
            pl.BlockSpec((1, _G), lambda t: (0, 0)),
            pl.BlockSpec((_B, _H), lambda t: (0, 0)),
            pl.BlockSpec((_B, _H), lambda t: (0, 0)),
        ],
        out_specs=[
            pl.BlockSpec((_B, _TC, _H),
                         lambda t: (0, jnp.maximum(t - 1, 0), 0)),
            pl.BlockSpec((2, _B, _H), lambda t: (0, 0, 0)),
            pl.BlockSpec((2, _B, _H), lambda t: (0, 0, 0)),
        ],
        out_shape=[
            jax.ShapeDtypeStruct((_B, _T, _H), jnp.float32),
            jax.ShapeDtypeStruct((2, _B, _H), jnp.float32),
            jax.ShapeDtypeStruct((2, _B, _H), jnp.float32),
        ],
        scratch_shapes=[
            pltpu.VMEM((_TC, _B, _G), jnp.float32),
            pltpu.VMEM((_TC, _B, _G), jnp.float32),
            pltpu.VMEM((2, _TC, _B, _H), jnp.float32),
            pltpu.VMEM((_TC, _B, _H), jnp.float32),
            pltpu.VMEM((_B, _H), jnp.float32),
            pltpu.VMEM((_B, _H), jnp.float32),
            pltpu.VMEM((_B, _H), jnp.float32),
            pltpu.VMEM((_B, _H), jnp.float32),
        ],
        compiler_params=pltpu.CompilerParams(
            dimension_semantics=("arbitrary",),
            vmem_limit_bytes=52 * 1024 * 1024,
        ),
        name="vd_lstm2",
    )(x, wih0T, whh0T, b0, wih1T, whh1T, b1, mask0, mask1)

    return out, (hn, cn)


# speedup vs baseline: 2.0821x; 1.1291x over previous
import jax
import jax.numpy as jnp
from jax import lax
from jax.experimental import pallas as pl
from jax.experimental.pallas import tpu as pltpu

_B, _T, _I, _H = 64, 2048, 128, 256
_TC = 32
_NT = _T // _TC
_G = 4 * _H


def _gates(g, c):
    i = jax.nn.sigmoid(g[:, 0 * _H:1 * _H])
    f = jax.nn.sigmoid(g[:, 1 * _H:2 * _H])
    gg = jnp.tanh(g[:, 2 * _H:3 * _H])
    o = jax.nn.sigmoid(g[:, 3 * _H:4 * _H])
    c_new = f * c + i * gg
    h_new = o * jnp.tanh(c_new)
    return h_new, c_new


def _lstm_kernel(x_ref, wih0_ref, whh0_ref, b0_ref, wih1_ref, whh1_ref,
                 b1_ref, m0_ref, m1_ref,
                 out_ref, hn_ref, cn_ref,
                 xw0_ref, xw1_ref, h1buf_ref, obuf_ref,
                 h0s, c0s, h1s, c1s):
    t = pl.program_id(0)
    cur = t % 2
    prev = (t + 1) % 2

    @pl.when(t == 0)
    def _():
        h0s[...] = jnp.zeros_like(h0s)
        c0s[...] = jnp.zeros_like(c0s)
        h1s[...] = jnp.zeros_like(h1s)
        c1s[...] = jnp.zeros_like(c1s)
        h1buf_ref[1] = jnp.zeros_like(h1buf_ref[1])

    hb = h1buf_ref[prev].reshape(_TC * _B, _H)
    xw1 = jnp.dot(hb, wih1_ref[...], preferred_element_type=jnp.float32)
    xw1_ref[...] = (xw1 + b1_ref[...]).reshape(_TC, _B, _G)

    xb = jnp.swapaxes(x_ref[...], 0, 1).reshape(_TC * _B, _I)
    xw0 = jnp.dot(xb, wih0_ref[...], preferred_element_type=jnp.float32)
    xw0_ref[...] = (xw0 + b0_ref[...]).reshape(_TC, _B, _G)

    whh0 = whh0_ref[...]
    whh1 = whh1_ref[...]
    m0 = m0_ref[...]
    m1 = m1_ref[...]

    def step(s, carry):
        h0, c0, h1, c1 = carry
        g0 = xw0_ref[s] + jnp.dot(h0, whh0, preferred_element_type=jnp.float32)
        g1 = xw1_ref[s] + jnp.dot(h1, whh1, preferred_element_type=jnp.float32)
        h0, c0 = _gates(g0, c0)
        h1, c1 = _gates(g1, c1)
        h1buf_ref[cur, s] = h0 * m0
        obuf_ref[s] = h1 * m1
        return (h0, c0, h1, c1)

    h0, c0, h1, c1 = lax.fori_loop(
        0, _TC, step, (h0s[...], c0s[...], h1s[...], c1s[...]), unroll=4)

    @pl.when(t < _NT)
    def _():
        h0s[...] = h0
        c0s[...] = c0

    @pl.when(t > 0)
    def _():
        h1s[...] = h1
        c1s[...] = c1

    out_ref[...] = jnp.swapaxes(obuf_ref[...], 0, 1)

    @pl.when(t == _NT)
    def _():
        hn_ref[0] = h0s[...]
        hn_ref[1] = h1
        cn_ref[0] = c0s[...]
        cn_ref[1] = c1


def kernel(x, W_ih0, W_hh0, b_ih0, b_hh0, W_ih1, W_hh1, b_ih1, b_hh1,
           mask0, mask1):
    wih0T = W_ih0.T
    whh0T = W_hh0.T
    b0 = (b_ih0 + b_hh0).reshape(1, _G)
    wih1T = W_ih1.T
    whh1T = W_hh1.T
    b1 = (b_ih1 + b_hh1).reshape(1, _G)

    out, hn, cn = pl.pallas_call(
        _lstm_kernel,
        grid=(_NT + 1,),
        in_specs=[
            pl.BlockSpec((_B, _TC, _I),
                         lambda t: (0, jnp.minimum(t, _NT - 1), 0)),
            pl.BlockSpec((_I, _G), lambda t: (0, 0)),
            pl.BlockSpec((_H, _G), lambda t: (0, 0)),
            pl.BlockSpec((1, _G), lambda t: (0, 0)),
            pl.BlockSpec((_H, _G), lambda t: (0, 0)),
            pl.BlockSpec((_H, _G), lambda t: (0, 0)),
            pl.BlockSpec((1, _G), lambda t: (0, 0)),
            pl.BlockSpec((_B, _H), lambda t: (0, 0)),
            pl.BlockSpec((_B, _H), lambda t: (0, 0)),
        ],
        out_specs=[
            pl.BlockSpec((_B, _TC, _H),
                         lambda t: (0, jnp.maximum(t - 1, 0), 0)),
            pl.BlockSpec((2, _B, _H), lambda t: (0, 0, 0)),
            pl.BlockSpec((2, _B, _H), lambda t: (0, 0, 0)),
        ],
        out_shape=[
            jax.ShapeDtypeStruct((_B, _T, _H), jnp.float32),
            jax.ShapeDtypeStruct((2, _B, _H), jnp.float32),
            jax.ShapeDtypeStruct((2, _B, _H), jnp.float32),
        ],
        scratch_shapes=[
            pltpu.VMEM((_TC, _B, _G), jnp.float32),
            pltpu.VMEM((_TC, _B, _G), jnp.float32),
            pltpu.VMEM((2, _TC, _B, _H), jnp.float32),
            pltpu.VMEM((_TC, _B, _H), jnp.float32),
            pltpu.VMEM((_B, _H), jnp.float32),
            pltpu.VMEM((_B, _H), jnp.float32),
            pltpu.VMEM((_B, _H), jnp.float32),
            pltpu.VMEM((_B, _H), jnp.float32),
        ],
        compiler_params=pltpu.CompilerParams(
            dimension_semantics=("arbitrary",),
            vmem_limit_bytes=52 * 1024 * 1024,
        ),
        name="vd_lstm2",
    )(x, wih0T, whh0T, b0, wih1T, whh1T, b1, mask0, mask1)

    return out, (hn, cn)


# speedup vs baseline: 2.2080x; 1.0605x over previous
import jax
import jax.numpy as jnp
from jax import lax
from jax.experimental import pallas as pl
from jax.experimental.pallas import tpu as pltpu

_B, _T, _I, _H = 64, 2048, 128, 256
_TC = 32
_NT = _T // _TC
_G = 4 * _H


def _gates(g, c):
    i = jax.nn.sigmoid(g[:, 0 * _H:1 * _H])
    f = jax.nn.sigmoid(g[:, 1 * _H:2 * _H])
    gg = jnp.tanh(g[:, 2 * _H:3 * _H])
    o = jax.nn.sigmoid(g[:, 3 * _H:4 * _H])
    c_new = f * c + i * gg
    h_new = o * jnp.tanh(c_new)
    return h_new, c_new


def _lstm_kernel(x_ref, wih0_ref, whh0_ref, b0_ref, wih1_ref, whh1_ref,
                 b1_ref, m0_ref, m1_ref,
                 out_ref, hn_ref, cn_ref,
                 xw0_ref, xw1_ref, h1buf_ref, obuf_ref,
                 h0s, c0s, h1s, c1s):
    t = pl.program_id(0)
    cur = t % 2
    prev = (t + 1) % 2

    @pl.when(t == 0)
    def _():
        h0s[...] = jnp.zeros_like(h0s)
        c0s[...] = jnp.zeros_like(c0s)
        h1s[...] = jnp.zeros_like(h1s)
        c1s[...] = jnp.zeros_like(c1s)
        h1buf_ref[1] = jnp.zeros_like(h1buf_ref[1])

    hb = h1buf_ref[prev].reshape(_TC * _B, _H)
    xw1 = jnp.dot(hb, wih1_ref[...], preferred_element_type=jnp.float32)
    xw1_ref[...] = (xw1 + b1_ref[...]).reshape(_TC, _B, _G)

    xb = jnp.swapaxes(x_ref[...], 0, 1).reshape(_TC * _B, _I)
    xw0 = jnp.dot(xb, wih0_ref[...], preferred_element_type=jnp.float32)
    xw0_ref[...] = (xw0 + b0_ref[...]).reshape(_TC, _B, _G)

    whh0 = whh0_ref[...]
    whh1 = whh1_ref[...]
    m0 = m0_ref[...]
    m1 = m1_ref[...]

    def step(s, carry):
        h0, c0, h1, c1 = carry
        g0 = xw0_ref[s] + jnp.dot(h0, whh0, preferred_element_type=jnp.float32)
        g1 = xw1_ref[s] + jnp.dot(h1, whh1, preferred_element_type=jnp.float32)
        h0, c0 = _gates(g0, c0)
        h1, c1 = _gates(g1, c1)
        h1buf_ref[cur, s] = h0 * m0
        obuf_ref[s] = h1 * m1
        return (h0, c0, h1, c1)

    h0, c0, h1, c1 = lax.fori_loop(
        0, _TC, step, (h0s[...], c0s[...], h1s[...], c1s[...]), unroll=8)

    @pl.when(t < _NT)
    def _():
        h0s[...] = h0
        c0s[...] = c0

    @pl.when(t > 0)
    def _():
        h1s[...] = h1
        c1s[...] = c1

    out_ref[...] = jnp.swapaxes(obuf_ref[...], 0, 1)

    @pl.when(t == _NT)
    def _():
        hn_ref[0] = h0s[...]
        hn_ref[1] = h1
        cn_ref[0] = c0s[...]
        cn_ref[1] = c1


def kernel(x, W_ih0, W_hh0, b_ih0, b_hh0, W_ih1, W_hh1, b_ih1, b_hh1,
           mask0, mask1):
    wih0T = W_ih0.T
    whh0T = W_hh0.T
    b0 = (b_ih0 + b_hh0).reshape(1, _G)
    wih1T = W_ih1.T
    whh1T = W_hh1.T
    b1 = (b_ih1 + b_hh1).reshape(1, _G)

    out, hn, cn = pl.pallas_call(
        _lstm_kernel,
        grid=(_NT + 1,),
        in_specs=[
            pl.BlockSpec((_B, _TC, _I),
                         lambda t: (0, jnp.minimum(t, _NT - 1), 0)),
            pl.BlockSpec((_I, _G), lambda t: (0, 0)),
            pl.BlockSpec((_H, _G), lambda t: (0, 0)),
            pl.BlockSpec((1, _G), lambda t: (0, 0)),
            pl.BlockSpec((_H, _G), lambda t: (0, 0)),
            pl.BlockSpec((_H, _G), lambda t: (0, 0)),
            pl.BlockSpec((1, _G), lambda t: (0, 0)),
            pl.BlockSpec((_B, _H), lambda t: (0, 0)),
            pl.BlockSpec((_B, _H), lambda t: (0, 0)),
        ],
        out_specs=[
            pl.BlockSpec((_B, _TC, _H),
                         lambda t: (0, jnp.maximum(t - 1, 0), 0)),
            pl.BlockSpec((2, _B, _H), lambda t: (0, 0, 0)),
            pl.BlockSpec((2, _B, _H), lambda t: (0, 0, 0)),
        ],
        out_shape=[
            jax.ShapeDtypeStruct((_B, _T, _H), jnp.float32),
            jax.ShapeDtypeStruct((2, _B, _H), jnp.float32),
            jax.ShapeDtypeStruct((2, _B, _H), jnp.float32),
        ],
        scratch_shapes=[
            pltpu.VMEM((_TC, _B, _G), jnp.float32),
            pltpu.VMEM((_TC, _B, _G), jnp.float32),
            pltpu.VMEM((2, _TC, _B, _H), jnp.float32),
            pltpu.VMEM((_TC, _B, _H), jnp.float32),
            pltpu.VMEM((_B, _H), jnp.float32),
            pltpu.VMEM((_B, _H), jnp.float32),
            pltpu.VMEM((_B, _H), jnp.float32),
            pltpu.VMEM((_B, _H), jnp.float32),
        ],
        compiler_params=pltpu.CompilerParams(
            dimension_semantics=("arbitrary",),
            vmem_limit_bytes=52 * 1024 * 1024,
        ),
        name="vd_lstm2",
    )(x, wih0T, whh0T, b0, wih1T, whh1T, b1, mask0, mask1)

    return out, (hn, cn)


# speedup vs baseline: 2.2718x; 1.0289x over previous
import jax
import jax.numpy as jnp
from jax import lax
from jax.experimental import pallas as pl
from jax.experimental.pallas import tpu as pltpu

_B, _T, _I, _H = 64, 2048, 128, 256
_TC = 32
_NT = _T // _TC
_G = 4 * _H


def _gates(g, c):
    i = jax.nn.sigmoid(g[:, 0 * _H:1 * _H])
    f = jax.nn.sigmoid(g[:, 1 * _H:2 * _H])
    gg = jnp.tanh(g[:, 2 * _H:3 * _H])
    o = jax.nn.sigmoid(g[:, 3 * _H:4 * _H])
    c_new = f * c + i * gg
    h_new = o * jnp.tanh(c_new)
    return h_new, c_new


def _lstm_kernel(x_ref, wih0_ref, whh0_ref, b0_ref, wih1_ref, whh1_ref,
                 b1_ref, m0_ref, m1_ref,
                 out_ref, hn_ref, cn_ref,
                 xw0_ref, xw1_ref, h1buf_ref, obuf_ref,
                 h0s, c0s, h1s, c1s):
    t = pl.program_id(0)
    cur = t % 2
    prev = (t + 1) % 2

    @pl.when(t == 0)
    def _():
        h0s[...] = jnp.zeros_like(h0s)
        c0s[...] = jnp.zeros_like(c0s)
        h1s[...] = jnp.zeros_like(h1s)
        c1s[...] = jnp.zeros_like(c1s)
        h1buf_ref[1] = jnp.zeros_like(h1buf_ref[1])

    hb = h1buf_ref[prev].reshape(_TC * _B, _H)
    xw1 = jnp.dot(hb, wih1_ref[...], preferred_element_type=jnp.float32)
    xw1_ref[...] = (xw1 + b1_ref[...]).reshape(_TC, _B, _G)

    xb = jnp.swapaxes(x_ref[...], 0, 1).reshape(_TC * _B, _I)
    xw0 = jnp.dot(xb, wih0_ref[...], preferred_element_type=jnp.float32)
    xw0_ref[...] = (xw0 + b0_ref[...]).reshape(_TC, _B, _G)

    whh0 = whh0_ref[...]
    whh1 = whh1_ref[...]
    m0 = m0_ref[...]
    m1 = m1_ref[...]

    def step(s, carry):
        h0, c0, h1, c1 = carry
        g0 = xw0_ref[s] + jnp.dot(h0, whh0, preferred_element_type=jnp.float32)
        g1 = xw1_ref[s] + jnp.dot(h1, whh1, preferred_element_type=jnp.float32)
        h0, c0 = _gates(g0, c0)
        h1, c1 = _gates(g1, c1)
        h1buf_ref[cur, s] = h0 * m0
        obuf_ref[s] = h1 * m1
        return (h0, c0, h1, c1)

    h0, c0, h1, c1 = lax.fori_loop(
        0, _TC, step, (h0s[...], c0s[...], h1s[...], c1s[...]), unroll=16)

    @pl.when(t < _NT)
    def _():
        h0s[...] = h0
        c0s[...] = c0

    @pl.when(t > 0)
    def _():
        h1s[...] = h1
        c1s[...] = c1

    out_ref[...] = jnp.swapaxes(obuf_ref[...], 0, 1)

    @pl.when(t == _NT)
    def _():
        hn_ref[0] = h0s[...]
        hn_ref[1] = h1
        cn_ref[0] = c0s[...]
        cn_ref[1] = c1


def kernel(x, W_ih0, W_hh0, b_ih0, b_hh0, W_ih1, W_hh1, b_ih1, b_hh1,
           mask0, mask1):
    wih0T = W_ih0.T
    whh0T = W_hh0.T
    b0 = (b_ih0 + b_hh0).reshape(1, _G)
    wih1T = W_ih1.T
    whh1T = W_hh1.T
    b1 = (b_ih1 + b_hh1).reshape(1, _G)

    out, hn, cn = pl.pallas_call(
        _lstm_kernel,
        grid=(_NT + 1,),
        in_specs=[
            pl.BlockSpec((_B, _TC, _I),
                         lambda t: (0, jnp.minimum(t, _NT - 1), 0)),
            pl.BlockSpec((_I, _G), lambda t: (0, 0)),
            pl.BlockSpec((_H, _G), lambda t: (0, 0)),
            pl.BlockSpec((1, _G), lambda t: (0, 0)),
            pl.BlockSpec((_H, _G), lambda t: (0, 0)),
            pl.BlockSpec((_H, _G), lambda t: (0, 0)),
            pl.BlockSpec((1, _G), lambda t: (0, 0)),
            pl.BlockSpec((_B, _H), lambda t: (0, 0)),
            pl.BlockSpec((_B, _H), lambda t: (0, 0)),
        ],
        out_specs=[
            pl.BlockSpec((_B, _TC, _H),
                         lambda t: (0, jnp.maximum(t - 1, 0), 0)),
            pl.BlockSpec((2, _B, _H), lambda t: (0, 0, 0)),
            pl.BlockSpec((2, _B, _H), lambda t: (0, 0, 0)),
        ],
        out_shape=[
            jax.ShapeDtypeStruct((_B, _T, _H), jnp.float32),
            jax.ShapeDtypeStruct((2, _B, _H), jnp.float32),
            jax.ShapeDtypeStruct((2, _B, _H), jnp.float32),
        ],
        scratch_shapes=[
            pltpu.VMEM((_TC, _B, _G), jnp.float32),
            pltpu.VMEM((_TC, _B, _G), jnp.float32),
            pltpu.VMEM((2, _TC, _B, _H), jnp.float32),
            pltpu.VMEM((_TC, _B, _H), jnp.float32),
            pltpu.VMEM((_B, _H), jnp.float32),
            pltpu.VMEM((_B, _H), jnp.float32),
            pltpu.VMEM((_B, _H), jnp.float32),
            pltpu.VMEM((_B, _H), jnp.float32),
        ],
        compiler_params=pltpu.CompilerParams(
            dimension_semantics=("arbitrary",),
            vmem_limit_bytes=52 * 1024 * 1024,
        ),
        name="vd_lstm2",
    )(x, wih0T, whh0T, b0, wih1T, whh1T, b1, mask0, mask1)

    return out, (hn, cn)


# speedup vs baseline: 2.3854x; 1.0500x over previous
import jax
import jax.numpy as jnp
from jax import lax
from jax.experimental import pallas as pl
from jax.experimental.pallas import tpu as pltpu

_B, _T, _I, _H = 64, 2048, 128, 256
_TC = 32
_NT = _T // _TC
_G = 4 * _H


def _gates(g, c):
    i = jax.nn.sigmoid(g[:, 0 * _H:1 * _H])
    f = jax.nn.sigmoid(g[:, 1 * _H:2 * _H])
    gg = jnp.tanh(g[:, 2 * _H:3 * _H])
    o = jax.nn.sigmoid(g[:, 3 * _H:4 * _H])
    c_new = f * c + i * gg
    h_new = o * jnp.tanh(c_new)
    return h_new, c_new


def _lstm_kernel(x_ref, wih0_ref, whh0_ref, b0_ref, wih1_ref, whh1_ref,
                 b1_ref, m0_ref, m1_ref,
                 out_ref, hn_ref, cn_ref,
                 xw0_ref, xw1_ref, h1buf_ref, obuf_ref,
                 h0s, c0s, h1s, c1s):
    t = pl.program_id(0)
    cur = t % 2
    prev = (t + 1) % 2

    @pl.when(t == 0)
    def _():
        h0s[...] = jnp.zeros_like(h0s)
        c0s[...] = jnp.zeros_like(c0s)
        h1s[...] = jnp.zeros_like(h1s)
        c1s[...] = jnp.zeros_like(c1s)
        h1buf_ref[1] = jnp.zeros_like(h1buf_ref[1])

    hb = h1buf_ref[prev].reshape(_TC * _B, _H)
    xw1 = jnp.dot(hb, wih1_ref[...], preferred_element_type=jnp.float32)
    xw1_ref[...] = (xw1 + b1_ref[...]).reshape(_TC, _B, _G)

    xb = jnp.swapaxes(x_ref[...], 0, 1).reshape(_TC * _B, _I)
    xw0 = jnp.dot(xb, wih0_ref[...], preferred_element_type=jnp.float32)
    xw0_ref[...] = (xw0 + b0_ref[...]).reshape(_TC, _B, _G)

    whh0 = whh0_ref[...]
    whh1 = whh1_ref[...]
    m0 = m0_ref[...]
    m1 = m1_ref[...]

    def step(s, carry):
        h0, c0, h1, c1 = carry
        g0 = xw0_ref[s] + jnp.dot(h0, whh0, preferred_element_type=jnp.float32)
        g1 = xw1_ref[s] + jnp.dot(h1, whh1, preferred_element_type=jnp.float32)
        h0, c0 = _gates(g0, c0)
        h1, c1 = _gates(g1, c1)
        h1buf_ref[cur, s] = h0 * m0
        obuf_ref[s] = h1 * m1
        return (h0, c0, h1, c1)

    h0, c0, h1, c1 = lax.fori_loop(
        0, _TC, step, (h0s[...], c0s[...], h1s[...], c1s[...]), unroll=32)

    @pl.when(t < _NT)
    def _():
        h0s[...] = h0
        c0s[...] = c0

    @pl.when(t > 0)
    def _():
        h1s[...] = h1
        c1s[...] = c1

    out_ref[...] = jnp.swapaxes(obuf_ref[...], 0, 1)

    @pl.when(t == _NT)
    def _():
        hn_ref[0] = h0s[...]
        hn_ref[1] = h1
        cn_ref[0] = c0s[...]
        cn_ref[1] = c1


def kernel(x, W_ih0, W_hh0, b_ih0, b_hh0, W_ih1, W_hh1, b_ih1, b_hh1,
           mask0, mask1):
    wih0T = W_ih0.T
    whh0T = W_hh0.T
    b0 = (b_ih0 + b_hh0).reshape(1, _G)
    wih1T = W_ih1.T
    whh1T = W_hh1.T
    b1 = (b_ih1 + b_hh1).reshape(1, _G)

    out, hn, cn = pl.pallas_call(
        _lstm_kernel,
        grid=(_NT + 1,),
        in_specs=[
            pl.BlockSpec((_B, _TC, _I),
                         lambda t: (0, jnp.minimum(t, _NT - 1), 0)),
            pl.BlockSpec((_I, _G), lambda t: (0, 0)),
            pl.BlockSpec((_H, _G), lambda t: (0, 0)),
            pl.BlockSpec((1, _G), lambda t: (0, 0)),
            pl.BlockSpec((_H, _G), lambda t: (0, 0)),
            pl.BlockSpec((_H, _G), lambda t: (0, 0)),
            pl.BlockSpec((1, _G), lambda t: (0, 0)),
            pl.BlockSpec((_B, _H), lambda t: (0, 0)),
            pl.BlockSpec((_B, _H), lambda t: (0, 0)),
        ],
        out_specs=[
            pl.BlockSpec((_B, _TC, _H),
                         lambda t: (0, jnp.maximum(t - 1, 0), 0)),
            pl.BlockSpec((2, _B, _H), lambda t: (0, 0, 0)),
            pl.BlockSpec((2, _B, _H), lambda t: (0, 0, 0)),
        ],
        out_shape=[
            jax.ShapeDtypeStruct((_B, _T, _H), jnp.float32),
            jax.ShapeDtypeStruct((2, _B, _H), jnp.float32),
            jax.ShapeDtypeStruct((2, _B, _H), jnp.float32),
        ],
        scratch_shapes=[
            pltpu.VMEM((_TC, _B, _G), jnp.float32),
            pltpu.VMEM((_TC, _B, _G), jnp.float32),
            pltpu.VMEM((2, _TC, _B, _H), jnp.float32),
            pltpu.VMEM((_TC, _B, _H), jnp.float32),
            pltpu.VMEM((_B, _H), jnp.float32),
            pltpu.VMEM((_B, _H), jnp.float32),
            pltpu.VMEM((_B, _H), jnp.float32),
            pltpu.VMEM((_B, _H), jnp.float32),
        ],
        compiler_params=pltpu.CompilerParams(
            dimension_semantics=("arbitrary",),
            vmem_limit_bytes=52 * 1024 * 1024,
        ),
        name="vd_lstm2",
    )(x, wih0T, whh0T, b0, wih1T, whh1T, b1, mask0, mask1)

    return out, (hn, cn)
